# baseline (speedup 1.0000x reference)
import functools

import jax
import jax.numpy as jnp
from jax import lax
from jax.experimental import pallas as pl
from jax.experimental.pallas import tpu as pltpu
from jax.experimental.pallas import tpu_sc as plsc

VOCAB = 100000
EMB = 16
CTX = 5
HID = 64
B = 1024

_info = plsc.get_sparse_core_info()
_NC, _NS = _info.num_cores, _info.num_subcores
_NW = _NC * _NS
_NIDX = B * CTX
_B_PER_W = _NIDX // _NW


def _gather_body(table_hbm, idx_hbm, out_hbm, idx_v, cols_v, sem):
    wid = lax.axis_index("s") * _NC + lax.axis_index("c")
    base = wid * _B_PER_W
    pltpu.sync_copy(idx_hbm.at[pl.ds(base, _B_PER_W)], idx_v)
    copies = [
        pltpu.async_copy(table_hbm.at[k].at[idx_v], cols_v.at[k], sem)
        for k in range(EMB)
    ]
    for c in copies:
        c.wait()
    pltpu.sync_copy(cols_v, out_hbm.at[:, pl.ds(base, _B_PER_W)])


_sc_gather = functools.partial(
    pl.kernel,
    mesh=plsc.VectorSubcoreMesh(core_axis_name="c", subcore_axis_name="s"),
    out_type=jax.ShapeDtypeStruct((EMB, _NIDX), jnp.float32),
    scratch_types=[
        pltpu.VMEM((_B_PER_W,), jnp.int32),
        pltpu.VMEM((EMB, _B_PER_W), jnp.float32),
        pltpu.SemaphoreType.DMA,
    ],
    compiler_params=pltpu.CompilerParams(use_tc_tiling_on_sc=False,
                                         skip_device_barrier=True),
)(_gather_body)


V_TILE = 2048


def _out_body(embsT_ref, W1_ref, b1_ref, W2_ref, b2_ref, out_ref, hid_ref):
    @pl.when(pl.program_id(0) == 0)
    def _():
        acc = jnp.broadcast_to(b1_ref[...], (B, HID))
        for j in range(CTX):
            acc = acc + jax.lax.dot_general(
                embsT_ref[:, j * B:(j + 1) * B],
                W1_ref[j * EMB:(j + 1) * EMB, :],
                (((0,), (0,)), ((), ())),
                preferred_element_type=jnp.float32)
        hid_ref[...] = jnp.maximum(acc, 0.0)

    acc = jax.lax.dot_general(
        W2_ref[...], hid_ref[...],
        (((0,), (1,)), ((), ())),
        preferred_element_type=jnp.float32)
    bias = jax.lax.broadcast_in_dim(
        b2_ref[...].reshape(V_TILE), (V_TILE, B), (0,))
    out_ref[...] = acc + bias


def _out_proj_t(embsT, W1, b1, W2, b2):
    nv = pl.cdiv(VOCAB, V_TILE)
    return pl.pallas_call(
        _out_body,
        grid=(nv,),
        in_specs=[
            pl.BlockSpec((EMB, CTX * B), lambda j: (0, 0)),
            pl.BlockSpec((CTX * EMB, HID), lambda j: (0, 0)),
            pl.BlockSpec((1, HID), lambda j: (0, 0)),
            pl.BlockSpec((HID, V_TILE), lambda j: (0, j)),
            pl.BlockSpec((1, V_TILE), lambda j: (0, j)),
        ],
        out_specs=pl.BlockSpec((V_TILE, B), lambda j: (j, 0)),
        out_shape=jax.ShapeDtypeStruct((VOCAB, B), jnp.float32),
        scratch_shapes=[pltpu.VMEM((B, HID), jnp.float32)],
    )(embsT, W1, b1, W2, b2)


def kernel(inputs, emb, W1, b1, W2, b2):
    idx = inputs.T.reshape(-1)
    embsT = _sc_gather(emb.T, idx)
    out_t = _out_proj_t(embsT, W1, b1.reshape(1, HID), W2,
                        b2.reshape(1, VOCAB))
    return out_t.T


# speedup vs baseline: 1.0116x; 1.0116x over previous
import functools

import jax
import jax.numpy as jnp
from jax import lax
from jax.experimental import pallas as pl
from jax.experimental.pallas import tpu as pltpu
from jax.experimental.pallas import tpu_sc as plsc

VOCAB = 100000
EMB = 16
CTX = 5
HID = 64
B = 1024

_info = plsc.get_sparse_core_info()
_NC, _NS = _info.num_cores, _info.num_subcores
_NW = _NC * _NS
_NIDX = B * CTX
_B_PER_W = _NIDX // _NW


def _gather_body(table_hbm, idx_hbm, out_hbm, idx_v, cols_v, sem):
    wid = lax.axis_index("s") * _NC + lax.axis_index("c")
    base = wid * _B_PER_W
    pltpu.sync_copy(idx_hbm.at[pl.ds(base, _B_PER_W)], idx_v)
    copies = [
        pltpu.async_copy(table_hbm.at[k].at[idx_v], cols_v.at[k], sem)
        for k in range(EMB)
    ]
    for c in copies:
        c.wait()
    pltpu.sync_copy(cols_v, out_hbm.at[:, pl.ds(base, _B_PER_W)])


_sc_gather = functools.partial(
    pl.kernel,
    mesh=plsc.VectorSubcoreMesh(core_axis_name="c", subcore_axis_name="s"),
    out_type=jax.ShapeDtypeStruct((EMB, _NIDX), jnp.float32),
    scratch_types=[
        pltpu.VMEM((_B_PER_W,), jnp.int32),
        pltpu.VMEM((EMB, _B_PER_W), jnp.float32),
        pltpu.SemaphoreType.DMA,
    ],
    compiler_params=pltpu.CompilerParams(use_tc_tiling_on_sc=False,
                                         skip_device_barrier=True),
)(_gather_body)


V_TILE = 4096


def _out_body(embsT_ref, W1_ref, b1_ref, W2_ref, b2_ref, out_ref, hid_ref):
    @pl.when(pl.program_id(0) == 0)
    def _():
        acc = jnp.broadcast_to(b1_ref[...], (B, HID))
        for j in range(CTX):
            acc = acc + jax.lax.dot_general(
                embsT_ref[:, j * B:(j + 1) * B],
                W1_ref[j * EMB:(j + 1) * EMB, :],
                (((0,), (0,)), ((), ())),
                preferred_element_type=jnp.float32)
        hid_ref[...] = jnp.maximum(acc, 0.0)

    acc = jax.lax.dot_general(
        W2_ref[...], hid_ref[...],
        (((0,), (1,)), ((), ())),
        preferred_element_type=jnp.float32)
    bias = jax.lax.broadcast_in_dim(
        b2_ref[...].reshape(V_TILE), (V_TILE, B), (0,))
    out_ref[...] = acc + bias


def _out_proj_t(embsT, W1, b1, W2, b2):
    nv = pl.cdiv(VOCAB, V_TILE)
    return pl.pallas_call(
        _out_body,
        grid=(nv,),
        in_specs=[
            pl.BlockSpec((EMB, CTX * B), lambda j: (0, 0)),
            pl.BlockSpec((CTX * EMB, HID), lambda j: (0, 0)),
            pl.BlockSpec((1, HID), lambda j: (0, 0)),
            pl.BlockSpec((HID, V_TILE), lambda j: (0, j)),
            pl.BlockSpec((1, V_TILE), lambda j: (0, j)),
        ],
        out_specs=pl.BlockSpec((V_TILE, B), lambda j: (j, 0)),
        out_shape=jax.ShapeDtypeStruct((VOCAB, B), jnp.float32),
        scratch_shapes=[pltpu.VMEM((B, HID), jnp.float32)],
    )(embsT, W1, b1, W2, b2)


def kernel(inputs, emb, W1, b1, W2, b2):
    idx = inputs.T.reshape(-1)
    embsT = _sc_gather(emb.T, idx)
    out_t = _out_proj_t(embsT, W1, b1.reshape(1, HID), W2,
                        b2.reshape(1, VOCAB))
    return out_t.T
